# baseline (speedup 1.0000x reference)
import functools

import jax
import jax.numpy as jnp
from jax import lax
from jax.experimental import pallas as pl
from jax.experimental.pallas import tpu as pltpu
from jax.experimental.pallas import tpu_sc as plsc

N = 10000
E = 320000
D = 128
NC = 2
NS = 16
NW = NC * NS
NP = 10240
EPT = E // NW
CH = 80
NCH = EPT // CH
SEG = NP // NS

_MESH = dict(core_axis_name="c", subcore_axis_name="s", num_cores=NC,
             num_subcores=NS)


def _zero_rows(ref, nrows):
    z = jnp.zeros((16,), jnp.float32)

    def body(r, _):
        for c in range(D // 16):
            ref[r, pl.ds(c * 16, 16)] = z
        return 0

    lax.fori_loop(0, nrows, body, 0)


NBUF = 5


def _sc_gather_body(x_hbm, src_hbm, dst_hbm, n2e_hbm, isv, idv, *rest):
    A = rest[0:NBUF]
    B = rest[NBUF:2 * NBUF]
    GA = rest[2 * NBUF:3 * NBUF]
    GB = rest[3 * NBUF:4 * NBUF]
    W = rest[4 * NBUF:5 * NBUF]
    cid = lax.axis_index("c")
    sid = lax.axis_index("s")
    wid = sid * NC + cid
    base = wid * EPT

    pltpu.sync_copy(src_hbm.at[pl.ds(base, EPT)], isv)
    pltpu.sync_copy(dst_hbm.at[pl.ds(base, EPT)], idv)

    def round_fn(t, _):
        for b in range(NBUF):
            @pl.when(t > 0)
            def _(b=b):
                pltpu.make_async_copy(
                    A[b], n2e_hbm.at[pl.ds(base, CH)], W[b]).wait()

            off = (t * NBUF + b) * CH
            pltpu.async_copy(x_hbm.at[isv.at[pl.ds(off, CH)]], A[b], GA[b])
            pltpu.async_copy(x_hbm.at[idv.at[pl.ds(off, CH)]], B[b], GB[b])
        for b in range(NBUF):
            pltpu.make_async_copy(
                x_hbm.at[isv.at[pl.ds(0, CH)]], A[b], GA[b]).wait()
            pltpu.make_async_copy(
                x_hbm.at[idv.at[pl.ds(0, CH)]], B[b], GB[b]).wait()

            def add_row(r, _, b=b):
                for c in range(D // 16):
                    sl = pl.ds(c * 16, 16)
                    A[b][r, sl] = A[b][r, sl] + B[b][r, sl]
                return 0

            lax.fori_loop(0, CH, add_row, 0)
            off = base + (t * NBUF + b) * CH
            pltpu.async_copy(A[b], n2e_hbm.at[pl.ds(off, CH)], W[b])
        return 0

    lax.fori_loop(0, NCH // NBUF, round_fn, 0)
    for b in range(NBUF):
        pltpu.make_async_copy(A[b], n2e_hbm.at[pl.ds(base, CH)], W[b]).wait()


def _sc_gather(x, src, dst, *, interpret=False):
    call = pl.kernel(
        _sc_gather_body,
        out_type=[jax.ShapeDtypeStruct((E, D), jnp.float32)],
        mesh=plsc.VectorSubcoreMesh(**_MESH),
        scratch_types=(
            [pltpu.VMEM((EPT,), jnp.int32)] * 2
            + [pltpu.VMEM((CH, D), jnp.float32)] * (2 * NBUF)
            + [pltpu.SemaphoreType.DMA] * (3 * NBUF)
        ),
        interpret=interpret,
    )
    return call(x, src, dst)[0]


DCH = 2000


def _sc_deg_body(src_hbm, dst_hbm, deg_hbm, idx_s, idx_d, deg_v, tmp_v,
                 shared_deg):
    cid = lax.axis_index("c")
    sid = lax.axis_index("s")
    wid = sid * NC + cid
    base = wid * EPT
    ones = jnp.ones((16,), jnp.float32)
    z = jnp.zeros((16,), jnp.float32)

    def zv(i, _):
        deg_v[pl.ds(i * 16, 16)] = z
        return 0

    lax.fori_loop(0, NP // 16, zv, 0)

    def chunk(j, _):
        off = base + j * DCH
        pltpu.sync_copy(src_hbm.at[pl.ds(off, DCH)], idx_s)
        pltpu.sync_copy(dst_hbm.at[pl.ds(off, DCH)], idx_d)

        def dg(i, _):
            s = idx_s[pl.ds(i * 16, 16)]
            plsc.addupdate_scatter(deg_v, [s], ones)
            d2 = idx_d[pl.ds(i * 16, 16)]
            plsc.addupdate_scatter(deg_v, [d2], ones)
            return 0

        lax.fori_loop(0, DCH // 16, dg, 0)
        return 0

    lax.fori_loop(0, EPT // DCH, chunk, 0)

    pltpu.sync_copy(deg_v, shared_deg.at[sid])
    plsc.subcore_barrier()

    def zv2(i, _):
        deg_v[pl.ds(i * 16, 16)] = z
        return 0

    lax.fori_loop(0, SEG // 16, zv2, 0)

    def red_row(r, _):
        pltpu.sync_copy(shared_deg.at[r, pl.ds(sid * SEG, SEG)], tmp_v)

        def addv(i, _):
            sl = pl.ds(i * 16, 16)
            deg_v[sl] = deg_v[sl] + tmp_v[sl]
            return 0

        lax.fori_loop(0, SEG // 16, addv, 0)
        return 0

    lax.fori_loop(0, NS, red_row, 0)
    pltpu.sync_copy(deg_v.at[pl.ds(0, SEG)],
                    deg_hbm.at[cid, pl.ds(sid * SEG, SEG)])


def _sc_deg(src, dst, *, interpret=False):
    call = pl.kernel(
        _sc_deg_body,
        out_type=[jax.ShapeDtypeStruct((NC, NP), jnp.float32)],
        mesh=plsc.VectorSubcoreMesh(**_MESH),
        scratch_types=[
            pltpu.VMEM((DCH,), jnp.int32),
            pltpu.VMEM((DCH,), jnp.int32),
            pltpu.VMEM((NP,), jnp.float32),
            pltpu.VMEM((SEG,), jnp.float32),
            pltpu.VMEM_SHARED((NS, NP), jnp.float32),
        ],
        compiler_params=pltpu.CompilerParams(needs_layout_passes=False),
        interpret=interpret,
    )
    return call(src, dst)[0]


SB_N = 4


def _sc_scatter_body(eo_hbm, src_hbm, dst_hbm, nb_hbm, *rest):
    RV = rest[0:SB_N]
    ISV = rest[SB_N:2 * SB_N]
    IDV = rest[2 * SB_N:3 * SB_N]
    LD = rest[3 * SB_N:4 * SB_N]
    LIS = rest[4 * SB_N:5 * SB_N]
    LID = rest[5 * SB_N:6 * SB_N]
    SA = rest[6 * SB_N:7 * SB_N]
    SB = rest[7 * SB_N:8 * SB_N]
    shared_nb = rest[8 * SB_N]
    cid = lax.axis_index("c")
    sid = lax.axis_index("s")
    wid = sid * NC + cid
    base = wid * EPT

    _zero_rows(RV[0], CH)
    for k in range(SEG // CH):
        pltpu.sync_copy(RV[0], shared_nb.at[pl.ds(sid * SEG + k * CH, CH)])
    plsc.subcore_barrier()

    def round_fn(t, _):
        for b in range(SB_N):
            off = base + (t * SB_N + b) * CH
            pltpu.async_copy(eo_hbm.at[pl.ds(off, CH)], RV[b], LD[b])
            pltpu.async_copy(src_hbm.at[pl.ds(off, CH)], ISV[b], LIS[b])
            pltpu.async_copy(dst_hbm.at[pl.ds(off, CH)], IDV[b], LID[b])
        for b in range(SB_N):
            pltpu.make_async_copy(
                eo_hbm.at[pl.ds(base, CH)], RV[b], LD[b]).wait()
            pltpu.make_async_copy(
                src_hbm.at[pl.ds(base, CH)], ISV[b], LIS[b]).wait()
            pltpu.make_async_copy(
                dst_hbm.at[pl.ds(base, CH)], IDV[b], LID[b]).wait()
            pltpu.sync_copy(RV[b], shared_nb.at[ISV[b]], add=True)
            pltpu.sync_copy(RV[b], shared_nb.at[IDV[b]], add=True)
        return 0

    nfull = NCH // SB_N
    lax.fori_loop(0, nfull, round_fn, 0)
    for j in range(nfull * SB_N, NCH):
        off = base + j * CH
        pltpu.sync_copy(eo_hbm.at[pl.ds(off, CH)], RV[0])
        pltpu.sync_copy(src_hbm.at[pl.ds(off, CH)], ISV[0])
        pltpu.sync_copy(dst_hbm.at[pl.ds(off, CH)], IDV[0])
        pltpu.sync_copy(RV[0], shared_nb.at[ISV[0]], add=True)
        pltpu.sync_copy(RV[0], shared_nb.at[IDV[0]], add=True)
    plsc.subcore_barrier()
    pltpu.sync_copy(shared_nb.at[pl.ds(sid * SEG, SEG)],
                    nb_hbm.at[cid, pl.ds(sid * SEG, SEG)])


def _sc_scatter(eo, src, dst, *, interpret=False):
    call = pl.kernel(
        _sc_scatter_body,
        out_type=[jax.ShapeDtypeStruct((NC, NP, D), jnp.float32)],
        mesh=plsc.VectorSubcoreMesh(**_MESH),
        scratch_types=(
            [pltpu.VMEM((CH, D), jnp.float32)] * SB_N
            + [pltpu.VMEM((CH,), jnp.int32)] * (2 * SB_N)
            + [pltpu.SemaphoreType.DMA] * (5 * SB_N)
            + [pltpu.VMEM_SHARED((NP, D), jnp.float32)]
        ),
        interpret=interpret,
    )
    return call(eo, src, dst)[0]


BE = 4000
_GE = E // BE


def _rep(shape):
    return pl.BlockSpec(shape, lambda i: tuple(0 for _ in shape))


def _dotbf(a, b):
    return jnp.dot(a.astype(jnp.bfloat16), b.astype(jnp.bfloat16),
                   preferred_element_type=jnp.float32)


def _acc_stats(acc_ref, h):
    @pl.when(pl.program_id(0) == 0)
    def _():
        acc_ref[...] = jnp.zeros_like(acc_ref)

    s = jnp.sum(h, axis=0, keepdims=True)
    s2 = jnp.sum(h * h, axis=0, keepdims=True)
    acc_ref[...] += jnp.concatenate([s, s2], axis=0)


def _pa_body(ea, n2e, w1a, w1b, b1, h1, acc):
    h = _dotbf(ea[...], w1a[...])
    h = h + _dotbf(n2e[...], w1b[...])
    h = h + b1[...]
    h1[...] = h.astype(jnp.bfloat16)
    _acc_stats(acc, h)


def _pass_a(ea, n2e, w1a, w1b, b1, *, interpret=False):
    return pl.pallas_call(
        _pa_body,
        grid=(_GE,),
        in_specs=[pl.BlockSpec((BE, D), lambda i: (i, 0)),
                  pl.BlockSpec((BE, D), lambda i: (i, 0)),
                  _rep((D, D)), _rep((D, D)), _rep((1, D))],
        out_specs=[pl.BlockSpec((BE, D), lambda i: (i, 0)),
                   _rep((2, D))],
        out_shape=[jax.ShapeDtypeStruct((E, D), jnp.bfloat16),
                   jax.ShapeDtypeStruct((2, D), jnp.float32)],
        interpret=interpret,
    )(ea, n2e, w1a, w1b, b1)


def _bn_affine(stats_ref, g_ref, be_ref):
    s0 = stats_ref[0:1, :]
    s1 = stats_ref[1:2, :]
    m = s0 * (1.0 / E)
    v = s1 * (1.0 / E) - m * m
    a = g_ref[...] * lax.rsqrt(v + 1e-5)
    c = be_ref[...] - m * a
    return a, c


def _pb_body(h1, stats, g1, be1, w2, b2, h2, acc):
    a, c = _bn_affine(stats, g1, be1)
    r = jnp.maximum(h1[...].astype(jnp.float32) * a + c, 0.0)
    h = _dotbf(r, w2[...]) + b2[...]
    h2[...] = h.astype(jnp.bfloat16)
    _acc_stats(acc, h)


def _pass_b(h1, st1, g1, be1, w2, b2, *, interpret=False):
    return pl.pallas_call(
        _pb_body,
        grid=(_GE,),
        in_specs=[pl.BlockSpec((BE, D), lambda i: (i, 0)),
                  _rep((2, D)), _rep((1, D)), _rep((1, D)),
                  _rep((D, D)), _rep((1, D))],
        out_specs=[pl.BlockSpec((BE, D), lambda i: (i, 0)),
                   _rep((2, D))],
        out_shape=[jax.ShapeDtypeStruct((E, D), jnp.bfloat16),
                   jax.ShapeDtypeStruct((2, D), jnp.float32)],
        interpret=interpret,
    )(h1, st1, g1, be1, w2, b2)


def _pc_body(h2, stats, g2, be2, ea, w1e, b1e, w2e, b2e, se, eo):
    a, c = _bn_affine(stats, g2, be2)
    g = jnp.maximum(h2[...].astype(jnp.float32) * a + c, 0.0)
    t = se[0, 0] * ea[...] + g
    r = jnp.maximum(_dotbf(t, w1e[...]) + b1e[...], 0.0)
    eo[...] = _dotbf(r, w2e[...]) + b2e[...]


def _pass_c(h2, st2, g2, be2, ea, w1e, b1e, w2e, b2e, se, *, interpret=False):
    return pl.pallas_call(
        _pc_body,
        grid=(_GE,),
        in_specs=[pl.BlockSpec((BE, D), lambda i: (i, 0)),
                  _rep((2, D)), _rep((1, D)), _rep((1, D)),
                  pl.BlockSpec((BE, D), lambda i: (i, 0)),
                  _rep((D, D)), _rep((1, D)), _rep((D, D)), _rep((1, D)),
                  pl.BlockSpec(memory_space=pltpu.SMEM)],
        out_specs=pl.BlockSpec((BE, D), lambda i: (i, 0)),
        out_shape=jax.ShapeDtypeStruct((E, D), jnp.float32),
        interpret=interpret,
    )(h2, st2, g2, be2, ea, w1e, b1e, w2e, b2e, se)


BN_BLK = 2000
_GN = N // BN_BLK


def _pd_body(x, nbp, degt, w1, b1, w2, b2, sn, out):
    xx = x[...]
    deg = degt[:, 0:1] + degt[:, 1:2]
    nb = nbp[0] + nbp[1] - xx * deg
    t = sn[0, 0] * xx + nb
    r = jnp.maximum(
        jnp.dot(t, w1[...], preferred_element_type=jnp.float32) + b1[...],
        0.0)
    out[...] = jnp.dot(r, w2[...], preferred_element_type=jnp.float32) \
        + b2[...]


def _pass_d(x, nbp, degt, w1, b1, w2, b2, sn, *, interpret=False):
    return pl.pallas_call(
        _pd_body,
        grid=(_GN,),
        in_specs=[pl.BlockSpec((BN_BLK, D), lambda i: (i, 0)),
                  pl.BlockSpec((NC, BN_BLK, D), lambda i: (0, i, 0)),
                  pl.BlockSpec((BN_BLK, NC), lambda i: (i, 0)),
                  _rep((D, D)), _rep((1, D)), _rep((D, D)), _rep((1, D)),
                  pl.BlockSpec(memory_space=pltpu.SMEM)],
        out_specs=pl.BlockSpec((BN_BLK, D), lambda i: (i, 0)),
        out_shape=jax.ShapeDtypeStruct((N, D), jnp.float32),
        interpret=interpret,
    )(x, nbp, degt, w1, b1, w2, b2, sn)


def kernel(x, edge_index, edge_attr, params):
    p = params
    src = edge_index[0]
    dst = edge_index[1]

    n2e = _sc_gather(x, src, dst)
    deg_parts = _sc_deg(src, dst)

    r1 = lambda a: a.reshape(1, D)
    h1, st1 = _pass_a(edge_attr, n2e, p['e1_W1'][:D], p['e1_W1'][D:],
                      r1(p['e1_b1']))
    h2, st2 = _pass_b(h1, st1, r1(p['e1_g1']), r1(p['e1_be1']),
                      p['e1_W2'], r1(p['e1_b2']))
    se = (1.0 + p['eps_edge']).reshape(1, 1)
    edge_out = _pass_c(h2, st2, r1(p['e1_g2']), r1(p['e1_be2']), edge_attr,
                       p['e2_W1'], r1(p['e2_b1']), p['e2_W2'], r1(p['e2_b2']),
                       se)

    nbp = _sc_scatter(edge_out, src, dst)
    sn = (1.0 + p['eps_node']).reshape(1, 1)
    node_out = _pass_d(x, nbp, deg_parts.T, p['n_W1'], r1(p['n_b1']),
                       p['n_W2'], r1(p['n_b2']), sn)
    return (node_out, edge_out)


# speedup vs baseline: 1.0816x; 1.0816x over previous
import functools

import jax
import jax.numpy as jnp
from jax import lax
from jax.experimental import pallas as pl
from jax.experimental.pallas import tpu as pltpu
from jax.experimental.pallas import tpu_sc as plsc

N = 10000
E = 320000
D = 128
NC = 2
NS = 16
NW = NC * NS
NP = 10240
EPT = E // NW
CH = 80
NCH = EPT // CH
SEG = NP // NS

_MESH = dict(core_axis_name="c", subcore_axis_name="s", num_cores=NC,
             num_subcores=NS)


def _zero_rows(ref, nrows):
    z = jnp.zeros((16,), jnp.float32)

    def body(r, _):
        for c in range(D // 16):
            ref[r, pl.ds(c * 16, 16)] = z
        return 0

    lax.fori_loop(0, nrows, body, 0)


NBUF = 5


def _sc_gather_body(x_hbm, src_hbm, dst_hbm, n2e_hbm, isv, idv, *rest):
    A = rest[0:NBUF]
    B = rest[NBUF:2 * NBUF]
    GA = rest[2 * NBUF:3 * NBUF]
    GB = rest[3 * NBUF:4 * NBUF]
    W = rest[4 * NBUF:5 * NBUF]
    cid = lax.axis_index("c")
    sid = lax.axis_index("s")
    wid = sid * NC + cid
    base = wid * EPT

    pltpu.sync_copy(src_hbm.at[pl.ds(base, EPT)], isv)
    pltpu.sync_copy(dst_hbm.at[pl.ds(base, EPT)], idv)

    def round_fn(t, _):
        for b in range(NBUF):
            @pl.when(t > 0)
            def _(b=b):
                pltpu.make_async_copy(
                    A[b], n2e_hbm.at[pl.ds(base, CH)], W[b]).wait()

            off = (t * NBUF + b) * CH
            pltpu.async_copy(x_hbm.at[isv.at[pl.ds(off, CH)]], A[b], GA[b])
            pltpu.async_copy(x_hbm.at[idv.at[pl.ds(off, CH)]], B[b], GB[b])
        for b in range(NBUF):
            pltpu.make_async_copy(
                x_hbm.at[isv.at[pl.ds(0, CH)]], A[b], GA[b]).wait()
            pltpu.make_async_copy(
                x_hbm.at[idv.at[pl.ds(0, CH)]], B[b], GB[b]).wait()

            def add_row(r, _, b=b):
                for c in range(D // 16):
                    sl = pl.ds(c * 16, 16)
                    A[b][r, sl] = A[b][r, sl] + B[b][r, sl]
                return 0

            lax.fori_loop(0, CH, add_row, 0)
            off = base + (t * NBUF + b) * CH
            pltpu.async_copy(A[b], n2e_hbm.at[pl.ds(off, CH)], W[b])
        return 0

    lax.fori_loop(0, NCH // NBUF, round_fn, 0)
    for b in range(NBUF):
        pltpu.make_async_copy(A[b], n2e_hbm.at[pl.ds(base, CH)], W[b]).wait()


def _sc_gather(x, src, dst, *, interpret=False):
    call = pl.kernel(
        _sc_gather_body,
        out_type=[jax.ShapeDtypeStruct((E, D), jnp.float32)],
        mesh=plsc.VectorSubcoreMesh(**_MESH),
        scratch_types=(
            [pltpu.VMEM((EPT,), jnp.int32)] * 2
            + [pltpu.VMEM((CH, D), jnp.float32)] * (2 * NBUF)
            + [pltpu.SemaphoreType.DMA] * (3 * NBUF)
        ),
        interpret=interpret,
    )
    return call(x, src, dst)[0]


DCH = 2000


def _sc_deg_body(src_hbm, dst_hbm, deg_hbm, idx_s, idx_d, deg_v, tmp_v,
                 shared_deg):
    cid = lax.axis_index("c")
    sid = lax.axis_index("s")
    wid = sid * NC + cid
    base = wid * EPT
    ones = jnp.ones((16,), jnp.float32)
    z = jnp.zeros((16,), jnp.float32)

    def zv(i, _):
        deg_v[pl.ds(i * 16, 16)] = z
        return 0

    lax.fori_loop(0, NP // 16, zv, 0)

    def chunk(j, _):
        off = base + j * DCH
        pltpu.sync_copy(src_hbm.at[pl.ds(off, DCH)], idx_s)
        pltpu.sync_copy(dst_hbm.at[pl.ds(off, DCH)], idx_d)

        def dg(i, _):
            s = idx_s[pl.ds(i * 16, 16)]
            plsc.addupdate_scatter(deg_v, [s], ones)
            d2 = idx_d[pl.ds(i * 16, 16)]
            plsc.addupdate_scatter(deg_v, [d2], ones)
            return 0

        lax.fori_loop(0, DCH // 16, dg, 0)
        return 0

    lax.fori_loop(0, EPT // DCH, chunk, 0)

    pltpu.sync_copy(deg_v, shared_deg.at[sid])
    plsc.subcore_barrier()

    def zv2(i, _):
        deg_v[pl.ds(i * 16, 16)] = z
        return 0

    lax.fori_loop(0, SEG // 16, zv2, 0)

    def red_row(r, _):
        pltpu.sync_copy(shared_deg.at[r, pl.ds(sid * SEG, SEG)], tmp_v)

        def addv(i, _):
            sl = pl.ds(i * 16, 16)
            deg_v[sl] = deg_v[sl] + tmp_v[sl]
            return 0

        lax.fori_loop(0, SEG // 16, addv, 0)
        return 0

    lax.fori_loop(0, NS, red_row, 0)
    pltpu.sync_copy(deg_v.at[pl.ds(0, SEG)],
                    deg_hbm.at[cid, pl.ds(sid * SEG, SEG)])


def _sc_deg(src, dst, *, interpret=False):
    call = pl.kernel(
        _sc_deg_body,
        out_type=[jax.ShapeDtypeStruct((NC, NP), jnp.float32)],
        mesh=plsc.VectorSubcoreMesh(**_MESH),
        scratch_types=[
            pltpu.VMEM((DCH,), jnp.int32),
            pltpu.VMEM((DCH,), jnp.int32),
            pltpu.VMEM((NP,), jnp.float32),
            pltpu.VMEM((SEG,), jnp.float32),
            pltpu.VMEM_SHARED((NS, NP), jnp.float32),
        ],
        compiler_params=pltpu.CompilerParams(needs_layout_passes=False),
        interpret=interpret,
    )
    return call(src, dst)[0]


SB_N = 4


def _sc_scatter_body(eo_hbm, src_hbm, dst_hbm, nb_hbm, *rest):
    RV = rest[0:SB_N]
    ISV = rest[SB_N:2 * SB_N]
    IDV = rest[2 * SB_N:3 * SB_N]
    LD = rest[3 * SB_N:4 * SB_N]
    LIS = rest[4 * SB_N:5 * SB_N]
    LID = rest[5 * SB_N:6 * SB_N]
    SA = rest[6 * SB_N:7 * SB_N]
    SB = rest[7 * SB_N:8 * SB_N]
    shared_nb = rest[8 * SB_N]
    cid = lax.axis_index("c")
    sid = lax.axis_index("s")
    wid = sid * NC + cid
    base = wid * EPT

    _zero_rows(RV[0], CH)
    for k in range(SEG // CH):
        pltpu.sync_copy(RV[0], shared_nb.at[pl.ds(sid * SEG + k * CH, CH)])
    plsc.subcore_barrier()

    def round_fn(t, _):
        for b in range(SB_N):
            off = base + (t * SB_N + b) * CH
            pltpu.async_copy(eo_hbm.at[pl.ds(off, CH)], RV[b], LD[b])
            pltpu.async_copy(src_hbm.at[pl.ds(off, CH)], ISV[b], LIS[b])
            pltpu.async_copy(dst_hbm.at[pl.ds(off, CH)], IDV[b], LID[b])
        for b in range(SB_N):
            pltpu.make_async_copy(
                eo_hbm.at[pl.ds(base, CH)], RV[b], LD[b]).wait()
            pltpu.make_async_copy(
                src_hbm.at[pl.ds(base, CH)], ISV[b], LIS[b]).wait()
            pltpu.make_async_copy(
                dst_hbm.at[pl.ds(base, CH)], IDV[b], LID[b]).wait()
            pltpu.sync_copy(RV[b], shared_nb.at[ISV[b]], add=True)
            pltpu.sync_copy(RV[b], shared_nb.at[IDV[b]], add=True)
        return 0

    nfull = NCH // SB_N
    lax.fori_loop(0, nfull, round_fn, 0)
    for j in range(nfull * SB_N, NCH):
        off = base + j * CH
        pltpu.sync_copy(eo_hbm.at[pl.ds(off, CH)], RV[0])
        pltpu.sync_copy(src_hbm.at[pl.ds(off, CH)], ISV[0])
        pltpu.sync_copy(dst_hbm.at[pl.ds(off, CH)], IDV[0])
        pltpu.sync_copy(RV[0], shared_nb.at[ISV[0]], add=True)
        pltpu.sync_copy(RV[0], shared_nb.at[IDV[0]], add=True)
    plsc.subcore_barrier()
    pltpu.sync_copy(shared_nb.at[pl.ds(sid * SEG, SEG)],
                    nb_hbm.at[cid, pl.ds(sid * SEG, SEG)])


def _sc_scatter(eo, src, dst, *, interpret=False):
    call = pl.kernel(
        _sc_scatter_body,
        out_type=[jax.ShapeDtypeStruct((NC, NP, D), jnp.float32)],
        mesh=plsc.VectorSubcoreMesh(**_MESH),
        scratch_types=(
            [pltpu.VMEM((CH, D), jnp.float32)] * SB_N
            + [pltpu.VMEM((CH,), jnp.int32)] * (2 * SB_N)
            + [pltpu.SemaphoreType.DMA] * (5 * SB_N)
            + [pltpu.VMEM_SHARED((NP, D), jnp.float32)]
        ),
        interpret=interpret,
    )
    return call(eo, src, dst)[0]


BE = 8000
_GE = E // BE


def _rep(shape):
    return pl.BlockSpec(shape, lambda i: tuple(0 for _ in shape))


def _acc_stats(acc_ref, h):
    @pl.when(pl.program_id(0) == 0)
    def _():
        acc_ref[...] = jnp.zeros_like(acc_ref)

    s = jnp.sum(h, axis=0, keepdims=True)
    s2 = jnp.sum(h * h, axis=0, keepdims=True)
    acc_ref[...] += jnp.concatenate([s, s2], axis=0)


def _pa_body(ea, n2e, w1a, w1b, b1, h1, acc):
    h = jnp.dot(ea[...], w1a[...], preferred_element_type=jnp.float32)
    h = h + jnp.dot(n2e[...], w1b[...], preferred_element_type=jnp.float32)
    h = h + b1[...]
    h1[...] = h.astype(jnp.bfloat16)
    _acc_stats(acc, h)


def _pass_a(ea, n2e, w1a, w1b, b1, *, interpret=False):
    return pl.pallas_call(
        _pa_body,
        grid=(_GE,),
        in_specs=[pl.BlockSpec((BE, D), lambda i: (i, 0)),
                  pl.BlockSpec((BE, D), lambda i: (i, 0)),
                  _rep((D, D)), _rep((D, D)), _rep((1, D))],
        out_specs=[pl.BlockSpec((BE, D), lambda i: (i, 0)),
                   _rep((2, D))],
        out_shape=[jax.ShapeDtypeStruct((E, D), jnp.bfloat16),
                   jax.ShapeDtypeStruct((2, D), jnp.float32)],
        interpret=interpret,
    )(ea, n2e, w1a, w1b, b1)


def _bn_affine(stats_ref, g_ref, be_ref):
    s0 = stats_ref[0:1, :]
    s1 = stats_ref[1:2, :]
    m = s0 * (1.0 / E)
    v = s1 * (1.0 / E) - m * m
    a = g_ref[...] * lax.rsqrt(v + 1e-5)
    c = be_ref[...] - m * a
    return a, c


def _pb_body(h1, stats, g1, be1, w2, b2, h2, acc):
    a, c = _bn_affine(stats, g1, be1)
    r = jnp.maximum(h1[...].astype(jnp.float32) * a + c, 0.0)
    h = jnp.dot(r, w2[...], preferred_element_type=jnp.float32) + b2[...]
    h2[...] = h.astype(jnp.bfloat16)
    _acc_stats(acc, h)


def _pass_b(h1, st1, g1, be1, w2, b2, *, interpret=False):
    return pl.pallas_call(
        _pb_body,
        grid=(_GE,),
        in_specs=[pl.BlockSpec((BE, D), lambda i: (i, 0)),
                  _rep((2, D)), _rep((1, D)), _rep((1, D)),
                  _rep((D, D)), _rep((1, D))],
        out_specs=[pl.BlockSpec((BE, D), lambda i: (i, 0)),
                   _rep((2, D))],
        out_shape=[jax.ShapeDtypeStruct((E, D), jnp.bfloat16),
                   jax.ShapeDtypeStruct((2, D), jnp.float32)],
        interpret=interpret,
    )(h1, st1, g1, be1, w2, b2)


def _pc_body(h2, stats, g2, be2, ea, w1e, b1e, w2e, b2e, se, eo):
    a, c = _bn_affine(stats, g2, be2)
    g = jnp.maximum(h2[...].astype(jnp.float32) * a + c, 0.0)
    t = se[0, 0] * ea[...] + g
    r = jnp.maximum(
        jnp.dot(t, w1e[...], preferred_element_type=jnp.float32) + b1e[...],
        0.0)
    eo[...] = jnp.dot(r, w2e[...], preferred_element_type=jnp.float32) \
        + b2e[...]


def _pass_c(h2, st2, g2, be2, ea, w1e, b1e, w2e, b2e, se, *, interpret=False):
    return pl.pallas_call(
        _pc_body,
        grid=(_GE,),
        in_specs=[pl.BlockSpec((BE, D), lambda i: (i, 0)),
                  _rep((2, D)), _rep((1, D)), _rep((1, D)),
                  pl.BlockSpec((BE, D), lambda i: (i, 0)),
                  _rep((D, D)), _rep((1, D)), _rep((D, D)), _rep((1, D)),
                  pl.BlockSpec(memory_space=pltpu.SMEM)],
        out_specs=pl.BlockSpec((BE, D), lambda i: (i, 0)),
        out_shape=jax.ShapeDtypeStruct((E, D), jnp.float32),
        interpret=interpret,
    )(h2, st2, g2, be2, ea, w1e, b1e, w2e, b2e, se)


BN_BLK = 2000
_GN = N // BN_BLK


def _pd_body(x, nbp, degt, w1, b1, w2, b2, sn, out):
    xx = x[...]
    deg = degt[:, 0:1] + degt[:, 1:2]
    nb = nbp[0] + nbp[1] - xx * deg
    t = sn[0, 0] * xx + nb
    r = jnp.maximum(
        jnp.dot(t, w1[...], preferred_element_type=jnp.float32) + b1[...],
        0.0)
    out[...] = jnp.dot(r, w2[...], preferred_element_type=jnp.float32) \
        + b2[...]


def _pass_d(x, nbp, degt, w1, b1, w2, b2, sn, *, interpret=False):
    return pl.pallas_call(
        _pd_body,
        grid=(_GN,),
        in_specs=[pl.BlockSpec((BN_BLK, D), lambda i: (i, 0)),
                  pl.BlockSpec((NC, BN_BLK, D), lambda i: (0, i, 0)),
                  pl.BlockSpec((BN_BLK, NC), lambda i: (i, 0)),
                  _rep((D, D)), _rep((1, D)), _rep((D, D)), _rep((1, D)),
                  pl.BlockSpec(memory_space=pltpu.SMEM)],
        out_specs=pl.BlockSpec((BN_BLK, D), lambda i: (i, 0)),
        out_shape=jax.ShapeDtypeStruct((N, D), jnp.float32),
        interpret=interpret,
    )(x, nbp, degt, w1, b1, w2, b2, sn)


def kernel(x, edge_index, edge_attr, params):
    p = params
    src = edge_index[0]
    dst = edge_index[1]

    n2e = _sc_gather(x, src, dst)
    deg_parts = _sc_deg(src, dst)

    r1 = lambda a: a.reshape(1, D)
    h1, st1 = _pass_a(edge_attr, n2e, p['e1_W1'][:D], p['e1_W1'][D:],
                      r1(p['e1_b1']))
    h2, st2 = _pass_b(h1, st1, r1(p['e1_g1']), r1(p['e1_be1']),
                      p['e1_W2'], r1(p['e1_b2']))
    se = (1.0 + p['eps_edge']).reshape(1, 1)
    edge_out = _pass_c(h2, st2, r1(p['e1_g2']), r1(p['e1_be2']), edge_attr,
                       p['e2_W1'], r1(p['e2_b1']), p['e2_W2'], r1(p['e2_b2']),
                       se)

    nbp = _sc_scatter(edge_out, src, dst)
    sn = (1.0 + p['eps_node']).reshape(1, 1)
    node_out = _pass_d(x, nbp, deg_parts.T, p['n_W1'], r1(p['n_b1']),
                       p['n_W2'], r1(p['n_b2']), sn)
    return (node_out, edge_out)


# speedup vs baseline: 1.1001x; 1.0171x over previous
import functools

import jax
import jax.numpy as jnp
from jax import lax
from jax.experimental import pallas as pl
from jax.experimental.pallas import tpu as pltpu
from jax.experimental.pallas import tpu_sc as plsc

N = 10000
E = 320000
D = 128
NC = 2
NS = 16
NW = NC * NS
NP = 10240
EPT = E // NW
CH = 80
NCH = EPT // CH
SEG = NP // NS

_MESH = dict(core_axis_name="c", subcore_axis_name="s", num_cores=NC,
             num_subcores=NS)


def _zero_rows(ref, nrows):
    z = jnp.zeros((16,), jnp.float32)

    def body(r, _):
        for c in range(D // 16):
            ref[r, pl.ds(c * 16, 16)] = z
        return 0

    lax.fori_loop(0, nrows, body, 0)


NBUF = 5


def _sc_gather_body(x_hbm, src_hbm, dst_hbm, n2e_hbm, isv, idv, *rest):
    A = rest[0:NBUF]
    B = rest[NBUF:2 * NBUF]
    GA = rest[2 * NBUF:3 * NBUF]
    GB = rest[3 * NBUF:4 * NBUF]
    W = rest[4 * NBUF:5 * NBUF]
    cid = lax.axis_index("c")
    sid = lax.axis_index("s")
    wid = sid * NC + cid
    base = wid * EPT

    pltpu.sync_copy(src_hbm.at[pl.ds(base, EPT)], isv)
    pltpu.sync_copy(dst_hbm.at[pl.ds(base, EPT)], idv)

    def round_fn(t, _):
        for b in range(NBUF):
            @pl.when(t > 0)
            def _(b=b):
                pltpu.make_async_copy(
                    A[b], n2e_hbm.at[pl.ds(base, CH)], W[b]).wait()

            off = (t * NBUF + b) * CH
            pltpu.async_copy(x_hbm.at[isv.at[pl.ds(off, CH)]], A[b], GA[b])
            pltpu.async_copy(x_hbm.at[idv.at[pl.ds(off, CH)]], B[b], GB[b])
        for b in range(NBUF):
            pltpu.make_async_copy(
                x_hbm.at[isv.at[pl.ds(0, CH)]], A[b], GA[b]).wait()
            pltpu.make_async_copy(
                x_hbm.at[idv.at[pl.ds(0, CH)]], B[b], GB[b]).wait()

            def add_row(r, _, b=b):
                for c in range(D // 16):
                    sl = pl.ds(c * 16, 16)
                    A[b][r, sl] = A[b][r, sl] + B[b][r, sl]
                return 0

            lax.fori_loop(0, CH, add_row, 0)
            off = base + (t * NBUF + b) * CH
            pltpu.async_copy(A[b], n2e_hbm.at[pl.ds(off, CH)], W[b])
        return 0

    lax.fori_loop(0, NCH // NBUF, round_fn, 0)
    for b in range(NBUF):
        pltpu.make_async_copy(A[b], n2e_hbm.at[pl.ds(base, CH)], W[b]).wait()


def _sc_gather(x, src, dst, *, interpret=False):
    call = pl.kernel(
        _sc_gather_body,
        out_type=[jax.ShapeDtypeStruct((E, D), jnp.float32)],
        mesh=plsc.VectorSubcoreMesh(**_MESH),
        scratch_types=(
            [pltpu.VMEM((EPT,), jnp.int32)] * 2
            + [pltpu.VMEM((CH, D), jnp.float32)] * (2 * NBUF)
            + [pltpu.SemaphoreType.DMA] * (3 * NBUF)
        ),
        interpret=interpret,
    )
    return call(x, src, dst)[0]


DCH = 2000


def _sc_deg_body(src_hbm, dst_hbm, deg_hbm, idx_s, idx_d, deg_v, tmp_v,
                 shared_deg):
    cid = lax.axis_index("c")
    sid = lax.axis_index("s")
    wid = sid * NC + cid
    base = wid * EPT
    ones = jnp.ones((16,), jnp.float32)
    z = jnp.zeros((16,), jnp.float32)

    def zv(i, _):
        deg_v[pl.ds(i * 16, 16)] = z
        return 0

    lax.fori_loop(0, NP // 16, zv, 0)

    def chunk(j, _):
        off = base + j * DCH
        pltpu.sync_copy(src_hbm.at[pl.ds(off, DCH)], idx_s)
        pltpu.sync_copy(dst_hbm.at[pl.ds(off, DCH)], idx_d)

        def dg(i, _):
            s = idx_s[pl.ds(i * 16, 16)]
            plsc.addupdate_scatter(deg_v, [s], ones)
            d2 = idx_d[pl.ds(i * 16, 16)]
            plsc.addupdate_scatter(deg_v, [d2], ones)
            return 0

        lax.fori_loop(0, DCH // 16, dg, 0)
        return 0

    lax.fori_loop(0, EPT // DCH, chunk, 0)

    pltpu.sync_copy(deg_v, shared_deg.at[sid])
    plsc.subcore_barrier()

    def zv2(i, _):
        deg_v[pl.ds(i * 16, 16)] = z
        return 0

    lax.fori_loop(0, SEG // 16, zv2, 0)

    def red_row(r, _):
        pltpu.sync_copy(shared_deg.at[r, pl.ds(sid * SEG, SEG)], tmp_v)

        def addv(i, _):
            sl = pl.ds(i * 16, 16)
            deg_v[sl] = deg_v[sl] + tmp_v[sl]
            return 0

        lax.fori_loop(0, SEG // 16, addv, 0)
        return 0

    lax.fori_loop(0, NS, red_row, 0)
    pltpu.sync_copy(deg_v.at[pl.ds(0, SEG)],
                    deg_hbm.at[cid, pl.ds(sid * SEG, SEG)])


def _sc_deg(src, dst, *, interpret=False):
    call = pl.kernel(
        _sc_deg_body,
        out_type=[jax.ShapeDtypeStruct((NC, NP), jnp.float32)],
        mesh=plsc.VectorSubcoreMesh(**_MESH),
        scratch_types=[
            pltpu.VMEM((DCH,), jnp.int32),
            pltpu.VMEM((DCH,), jnp.int32),
            pltpu.VMEM((NP,), jnp.float32),
            pltpu.VMEM((SEG,), jnp.float32),
            pltpu.VMEM_SHARED((NS, NP), jnp.float32),
        ],
        compiler_params=pltpu.CompilerParams(needs_layout_passes=False),
        interpret=interpret,
    )
    return call(src, dst)[0]


SB_N = 4


def _sc_scatter_body(eo_hbm, src_hbm, dst_hbm, nb_hbm, *rest):
    RV = rest[0:SB_N]
    ISV = rest[SB_N:2 * SB_N]
    IDV = rest[2 * SB_N:3 * SB_N]
    LD = rest[3 * SB_N:4 * SB_N]
    LIS = rest[4 * SB_N:5 * SB_N]
    LID = rest[5 * SB_N:6 * SB_N]
    SA = rest[6 * SB_N:7 * SB_N]
    SB = rest[7 * SB_N:8 * SB_N]
    shared_nb = rest[8 * SB_N]
    cid = lax.axis_index("c")
    sid = lax.axis_index("s")
    wid = sid * NC + cid
    base = wid * EPT

    _zero_rows(RV[0], CH)
    for k in range(SEG // CH):
        pltpu.sync_copy(RV[0], shared_nb.at[pl.ds(sid * SEG + k * CH, CH)])
    plsc.subcore_barrier()

    def round_fn(t, _):
        for b in range(SB_N):
            off = base + (t * SB_N + b) * CH
            pltpu.async_copy(eo_hbm.at[pl.ds(off, CH)], RV[b], LD[b])
            pltpu.async_copy(src_hbm.at[pl.ds(off, CH)], ISV[b], LIS[b])
            pltpu.async_copy(dst_hbm.at[pl.ds(off, CH)], IDV[b], LID[b])
        for b in range(SB_N):
            pltpu.make_async_copy(
                eo_hbm.at[pl.ds(base, CH)], RV[b], LD[b]).wait()
            pltpu.make_async_copy(
                src_hbm.at[pl.ds(base, CH)], ISV[b], LIS[b]).wait()
            pltpu.make_async_copy(
                dst_hbm.at[pl.ds(base, CH)], IDV[b], LID[b]).wait()
            pltpu.sync_copy(RV[b], shared_nb.at[ISV[b]], add=True)
            pltpu.sync_copy(RV[b], shared_nb.at[IDV[b]], add=True)
        return 0

    nfull = NCH // SB_N
    lax.fori_loop(0, nfull, round_fn, 0)
    for j in range(nfull * SB_N, NCH):
        off = base + j * CH
        pltpu.sync_copy(eo_hbm.at[pl.ds(off, CH)], RV[0])
        pltpu.sync_copy(src_hbm.at[pl.ds(off, CH)], ISV[0])
        pltpu.sync_copy(dst_hbm.at[pl.ds(off, CH)], IDV[0])
        pltpu.sync_copy(RV[0], shared_nb.at[ISV[0]], add=True)
        pltpu.sync_copy(RV[0], shared_nb.at[IDV[0]], add=True)
    plsc.subcore_barrier()
    pltpu.sync_copy(shared_nb.at[pl.ds(sid * SEG, SEG)],
                    nb_hbm.at[cid, pl.ds(sid * SEG, SEG)])


def _sc_scatter(eo, src, dst, *, interpret=False):
    call = pl.kernel(
        _sc_scatter_body,
        out_type=[jax.ShapeDtypeStruct((NC, NP, D), jnp.float32)],
        mesh=plsc.VectorSubcoreMesh(**_MESH),
        scratch_types=(
            [pltpu.VMEM((CH, D), jnp.float32)] * SB_N
            + [pltpu.VMEM((CH,), jnp.int32)] * (2 * SB_N)
            + [pltpu.SemaphoreType.DMA] * (5 * SB_N)
            + [pltpu.VMEM_SHARED((NP, D), jnp.float32)]
        ),
        interpret=interpret,
    )
    return call(eo, src, dst)[0]


BE = 16000
_GE = E // BE


def _rep(shape):
    return pl.BlockSpec(shape, lambda i: tuple(0 for _ in shape))


def _acc_stats(acc_ref, h):
    @pl.when(pl.program_id(0) == 0)
    def _():
        acc_ref[...] = jnp.zeros_like(acc_ref)

    s = jnp.sum(h, axis=0, keepdims=True)
    s2 = jnp.sum(h * h, axis=0, keepdims=True)
    acc_ref[...] += jnp.concatenate([s, s2], axis=0)


def _pa_body(ea, n2e, w1a, w1b, b1, h1, acc):
    h = jnp.dot(ea[...], w1a[...], preferred_element_type=jnp.float32)
    h = h + jnp.dot(n2e[...], w1b[...], preferred_element_type=jnp.float32)
    h = h + b1[...]
    h1[...] = h.astype(jnp.bfloat16)
    _acc_stats(acc, h)


def _pass_a(ea, n2e, w1a, w1b, b1, *, interpret=False):
    return pl.pallas_call(
        _pa_body,
        grid=(_GE,),
        in_specs=[pl.BlockSpec((BE, D), lambda i: (i, 0)),
                  pl.BlockSpec((BE, D), lambda i: (i, 0)),
                  _rep((D, D)), _rep((D, D)), _rep((1, D))],
        out_specs=[pl.BlockSpec((BE, D), lambda i: (i, 0)),
                   _rep((2, D))],
        out_shape=[jax.ShapeDtypeStruct((E, D), jnp.bfloat16),
                   jax.ShapeDtypeStruct((2, D), jnp.float32)],
        interpret=interpret,
    )(ea, n2e, w1a, w1b, b1)


def _bn_affine(stats_ref, g_ref, be_ref):
    s0 = stats_ref[0:1, :]
    s1 = stats_ref[1:2, :]
    m = s0 * (1.0 / E)
    v = s1 * (1.0 / E) - m * m
    a = g_ref[...] * lax.rsqrt(v + 1e-5)
    c = be_ref[...] - m * a
    return a, c


def _pb_body(h1, stats, g1, be1, w2, b2, h2, acc):
    a, c = _bn_affine(stats, g1, be1)
    r = jnp.maximum(h1[...].astype(jnp.float32) * a + c, 0.0)
    h = jnp.dot(r, w2[...], preferred_element_type=jnp.float32) + b2[...]
    h2[...] = h.astype(jnp.bfloat16)
    _acc_stats(acc, h)


def _pass_b(h1, st1, g1, be1, w2, b2, *, interpret=False):
    return pl.pallas_call(
        _pb_body,
        grid=(_GE,),
        in_specs=[pl.BlockSpec((BE, D), lambda i: (i, 0)),
                  _rep((2, D)), _rep((1, D)), _rep((1, D)),
                  _rep((D, D)), _rep((1, D))],
        out_specs=[pl.BlockSpec((BE, D), lambda i: (i, 0)),
                   _rep((2, D))],
        out_shape=[jax.ShapeDtypeStruct((E, D), jnp.bfloat16),
                   jax.ShapeDtypeStruct((2, D), jnp.float32)],
        interpret=interpret,
    )(h1, st1, g1, be1, w2, b2)


def _pc_body(h2, stats, g2, be2, ea, w1e, b1e, w2e, b2e, se, eo):
    a, c = _bn_affine(stats, g2, be2)
    g = jnp.maximum(h2[...].astype(jnp.float32) * a + c, 0.0)
    t = se[0, 0] * ea[...] + g
    r = jnp.maximum(
        jnp.dot(t, w1e[...], preferred_element_type=jnp.float32) + b1e[...],
        0.0)
    eo[...] = jnp.dot(r, w2e[...], preferred_element_type=jnp.float32) \
        + b2e[...]


def _pass_c(h2, st2, g2, be2, ea, w1e, b1e, w2e, b2e, se, *, interpret=False):
    return pl.pallas_call(
        _pc_body,
        grid=(_GE,),
        in_specs=[pl.BlockSpec((BE, D), lambda i: (i, 0)),
                  _rep((2, D)), _rep((1, D)), _rep((1, D)),
                  pl.BlockSpec((BE, D), lambda i: (i, 0)),
                  _rep((D, D)), _rep((1, D)), _rep((D, D)), _rep((1, D)),
                  pl.BlockSpec(memory_space=pltpu.SMEM)],
        out_specs=pl.BlockSpec((BE, D), lambda i: (i, 0)),
        out_shape=jax.ShapeDtypeStruct((E, D), jnp.float32),
        interpret=interpret,
    )(h2, st2, g2, be2, ea, w1e, b1e, w2e, b2e, se)


BN_BLK = 2000
_GN = N // BN_BLK


def _pd_body(x, nbp, degt, w1, b1, w2, b2, sn, out):
    xx = x[...]
    deg = degt[:, 0:1] + degt[:, 1:2]
    nb = nbp[0] + nbp[1] - xx * deg
    t = sn[0, 0] * xx + nb
    r = jnp.maximum(
        jnp.dot(t, w1[...], preferred_element_type=jnp.float32) + b1[...],
        0.0)
    out[...] = jnp.dot(r, w2[...], preferred_element_type=jnp.float32) \
        + b2[...]


def _pass_d(x, nbp, degt, w1, b1, w2, b2, sn, *, interpret=False):
    return pl.pallas_call(
        _pd_body,
        grid=(_GN,),
        in_specs=[pl.BlockSpec((BN_BLK, D), lambda i: (i, 0)),
                  pl.BlockSpec((NC, BN_BLK, D), lambda i: (0, i, 0)),
                  pl.BlockSpec((BN_BLK, NC), lambda i: (i, 0)),
                  _rep((D, D)), _rep((1, D)), _rep((D, D)), _rep((1, D)),
                  pl.BlockSpec(memory_space=pltpu.SMEM)],
        out_specs=pl.BlockSpec((BN_BLK, D), lambda i: (i, 0)),
        out_shape=jax.ShapeDtypeStruct((N, D), jnp.float32),
        interpret=interpret,
    )(x, nbp, degt, w1, b1, w2, b2, sn)


def kernel(x, edge_index, edge_attr, params):
    p = params
    src = edge_index[0]
    dst = edge_index[1]

    n2e = _sc_gather(x, src, dst)
    deg_parts = _sc_deg(src, dst)

    r1 = lambda a: a.reshape(1, D)
    h1, st1 = _pass_a(edge_attr, n2e, p['e1_W1'][:D], p['e1_W1'][D:],
                      r1(p['e1_b1']))
    h2, st2 = _pass_b(h1, st1, r1(p['e1_g1']), r1(p['e1_be1']),
                      p['e1_W2'], r1(p['e1_b2']))
    se = (1.0 + p['eps_edge']).reshape(1, 1)
    edge_out = _pass_c(h2, st2, r1(p['e1_g2']), r1(p['e1_be2']), edge_attr,
                       p['e2_W1'], r1(p['e2_b1']), p['e2_W2'], r1(p['e2_b2']),
                       se)

    nbp = _sc_scatter(edge_out, src, dst)
    sn = (1.0 + p['eps_node']).reshape(1, 1)
    node_out = _pass_d(x, nbp, deg_parts.T, p['n_W1'], r1(p['n_b1']),
                       p['n_W2'], r1(p['n_b2']), sn)
    return (node_out, edge_out)
